# Initial kernel scaffold; baseline (speedup 1.0000x reference)
import jax
import jax.numpy as jnp
from jax.experimental import pallas as pl


def kernel(target, context, neg_samples, emb_target, emb_context):
    raise NotImplementedError("write your pallas kernel here")


# speedup vs baseline: 4.7197x; 4.7197x over previous
import functools

import jax
import jax.numpy as jnp
from jax import lax
from jax.experimental import pallas as pl
from jax.experimental.pallas import tpu as pltpu
from jax.experimental.pallas import tpu_sc as plsc

D = 64
B = 16384
L = 20
K = 20
NC = 2
NS = 16
NW = NC * NS
EPW = B // NW
CH = 32
NCHUNK = EPW // CH

_mesh = plsc.VectorSubcoreMesh(core_axis_name="c", subcore_axis_name="s")


@functools.partial(
    pl.kernel,
    out_type=[
        jax.ShapeDtypeStruct((B,), jnp.float32),
        jax.ShapeDtypeStruct((B * K,), jnp.float32),
    ],
    mesh=_mesh,
    compiler_params=pltpu.CompilerParams(
        needs_layout_passes=False, use_tc_tiling_on_sc=False),
    scratch_types=[
        pltpu.VMEM((CH,), jnp.int32),
        pltpu.VMEM((CH * L,), jnp.int32),
        pltpu.VMEM((CH * K,), jnp.int32),
        pltpu.VMEM((CH, D), jnp.float32),
        pltpu.VMEM((CH * L, D), jnp.float32),
        pltpu.VMEM((CH * K, D), jnp.float32),
        pltpu.VMEM((512,), jnp.float32),
        pltpu.VMEM((CH,), jnp.float32),
        pltpu.VMEM((CH * K,), jnp.float32),
        pltpu.SemaphoreType.DMA,
        pltpu.SemaphoreType.DMA,
        pltpu.SemaphoreType.DMA,
    ],
)
def _sc_scores(tgt_hbm, ctx_hbm, neg_hbm, embt_hbm, embc_hbm,
               pos_hbm, negout_hbm,
               tgt_idx, ctx_idx, neg_idx,
               t_rows, c_rows, n_rows,
               pbuf, posbuf, negbuf,
               sem_t, sem_c, sem_n):
    wid = lax.axis_index("s") * NC + lax.axis_index("c")
    base = wid * EPW
    iota = lax.iota(jnp.int32, 16)
    rowsel = iota * 16
    mask4 = iota < 4
    mask_pos = iota == 4

    def chunk_body(g, carry):
        eb = base + g * CH
        pltpu.sync_copy(tgt_hbm.at[pl.ds(eb, CH)], tgt_idx)
        pltpu.sync_copy(ctx_hbm.at[pl.ds(eb * L, CH * L)], ctx_idx)
        pltpu.sync_copy(neg_hbm.at[pl.ds(eb * K, CH * K)], neg_idx)
        cpt = pltpu.async_copy(embt_hbm.at[tgt_idx], t_rows, sem_t)
        cpc = pltpu.async_copy(embc_hbm.at[ctx_idx], c_rows, sem_c)
        cpn = pltpu.async_copy(embc_hbm.at[neg_idx], n_rows, sem_n)
        cpt.wait()
        cpc.wait()
        cpn.wait()

        def elem_body(e, inner):
            c0 = c_rows[e * L, pl.ds(0, 16)]
            c1 = c_rows[e * L, pl.ds(16, 16)]
            c2 = c_rows[e * L, pl.ds(32, 16)]
            c3 = c_rows[e * L, pl.ds(48, 16)]
            for l in range(1, L):
                c0 = c0 + c_rows[e * L + l, pl.ds(0, 16)]
                c1 = c1 + c_rows[e * L + l, pl.ds(16, 16)]
                c2 = c2 + c_rows[e * L + l, pl.ds(32, 16)]
                c3 = c3 + c_rows[e * L + l, pl.ds(48, 16)]
            scale = jnp.float32(1.0 / L)
            c0 = c0 * scale
            c1 = c1 * scale
            c2 = c2 * scale
            c3 = c3 * scale
            for k in range(K):
                r = e * K + k
                p = (n_rows[r, pl.ds(0, 16)] * c0
                     + n_rows[r, pl.ds(16, 16)] * c1
                     + n_rows[r, pl.ds(32, 16)] * c2
                     + n_rows[r, pl.ds(48, 16)] * c3)
                pbuf[pl.ds(k * 16, 16)] = p
            p = (t_rows[e, pl.ds(0, 16)] * c0
                 + t_rows[e, pl.ds(16, 16)] * c1
                 + t_rows[e, pl.ds(32, 16)] * c2
                 + t_rows[e, pl.ds(48, 16)] * c3)
            pbuf[pl.ds(K * 16, 16)] = p
            acc = plsc.load_gather(pbuf, [rowsel])
            for j in range(1, 16):
                acc = acc + plsc.load_gather(pbuf, [rowsel + j])
            plsc.store_scatter(negbuf, [e * K + iota], acc)
            acc2 = plsc.load_gather(pbuf, [rowsel + 256])
            for j in range(1, 16):
                acc2 = acc2 + plsc.load_gather(pbuf, [rowsel + 256 + j])
            plsc.store_scatter(negbuf, [e * K + 16 + iota], acc2, mask=mask4)
            plsc.store_scatter(posbuf, [iota * 0 + e], acc2, mask=mask_pos)
            return inner

        lax.fori_loop(0, CH, elem_body, 0)
        pltpu.sync_copy(posbuf, pos_hbm.at[pl.ds(eb, CH)])
        pltpu.sync_copy(negbuf, negout_hbm.at[pl.ds(eb * K, CH * K)])
        return carry

    lax.fori_loop(0, NCHUNK, chunk_body, 0)


def _loss_body(pos_ref, neg_ref, out_ref):
    p = pos_ref[...]
    n = neg_ref[...]
    lp = jnp.maximum(-p, 0.0) + jnp.log1p(jnp.exp(-jnp.abs(p)))
    ln = jnp.maximum(n, 0.0) + jnp.log1p(jnp.exp(-jnp.abs(n)))
    out_ref[0, 0] = (jnp.sum(lp) + jnp.sum(ln)) * jnp.float32(1.0 / B)


_loss_call = pl.pallas_call(
    _loss_body,
    out_shape=jax.ShapeDtypeStruct((1, 1), jnp.float32),
    out_specs=pl.BlockSpec(memory_space=pltpu.SMEM),
)


def kernel(target, context, neg_samples, emb_target, emb_context):
    tgt = target.astype(jnp.int32)
    ctx = context.astype(jnp.int32).reshape(-1)
    neg = neg_samples.astype(jnp.int32).reshape(-1)
    pos, negs = _sc_scores(tgt, ctx, neg, emb_target, emb_context)
    loss = _loss_call(pos.reshape(128, 128), negs.reshape(B * K // 128, 128))
    return loss[0, 0]
